# Initial kernel scaffold; baseline (speedup 1.0000x reference)
import jax
import jax.numpy as jnp
from jax.experimental import pallas as pl


def kernel(Z, R, e_source, e_target, molecule_ind, total_charge, params):
    raise NotImplementedError("write your pallas kernel here")


# speedup vs baseline: 25.6370x; 25.6370x over previous
import functools

import numpy as np
import jax
import jax.numpy as jnp
from jax import lax
from jax.experimental import pallas as pl
from jax.experimental.pallas import tpu as pltpu
from jax.experimental.pallas import tpu_sc as plsc

N_MESSAGE = 3
N_RBF = 8
N_EMBED = 8
R_CUT = 5.0
MAX_Z = 35
NATOM = 10000
NEDGE = 160000
NMOL = 500

NW = 32
CHUNK = 128
SCH = 40
EPAD = NW * SCH * CHUNK
ET = 1024
AT = 2000
W_ACC = 144
W_INV = 96

_PERM = np.array(
    list(range(32)) + list(range(288, 296))
    + [32 + a * 8 + r for r in range(8) for a in range(32)], dtype=np.int32)

_CENTERS = np.linspace(0.0, R_CUT, N_RBF).astype(np.float32)


def _sc_gather(table, idx):
  B = idx.shape[0]
  nch = B // (NW * CHUNK)
  idx3 = idx.reshape(NW, nch, CHUNK)
  mesh = plsc.VectorSubcoreMesh(core_axis_name="c", subcore_axis_name="s")

  @functools.partial(
      pl.kernel, mesh=mesh,
      compiler_params=pltpu.CompilerParams(use_tc_tiling_on_sc=False),
      out_type=jax.ShapeDtypeStruct((NW, nch, CHUNK, 16), jnp.float32),
      scratch_types=[
          pltpu.VMEM((nch, CHUNK), jnp.int32),
          pltpu.VMEM((CHUNK, 16), jnp.float32),
          pltpu.SemaphoreType.DMA,
      ],
  )
  def k(table_hbm, idx_hbm, out_hbm, idx_v, rows_v, sem):
    cid = lax.axis_index("c")
    sid = lax.axis_index("s")
    wid = sid * 2 + cid
    pltpu.sync_copy(idx_hbm.at[wid], idx_v)

    def body(ci, carry):
      pltpu.async_copy(table_hbm.at[idx_v.at[ci]], rows_v, sem).wait()
      pltpu.sync_copy(rows_v, out_hbm.at[wid, ci])
      return carry

    lax.fori_loop(0, nch, body, 0)

  return k(table, idx3).reshape(B, 16)


def _sc_scatter(payload, idx, width):
  pay4 = payload.reshape(NW, SCH, CHUNK, width)
  idx3 = idx.reshape(NW, SCH, CHUNK)
  zeros = jnp.zeros((NATOM, width), jnp.float32)
  rows = NATOM // 16
  mesh = plsc.VectorSubcoreMesh(core_axis_name="c", subcore_axis_name="s")

  @functools.partial(
      pl.kernel, mesh=mesh,
      compiler_params=pltpu.CompilerParams(use_tc_tiling_on_sc=False),
      out_type=jax.ShapeDtypeStruct((2, NATOM, width), jnp.float32),
      scratch_types=[
          pltpu.VMEM((SCH, CHUNK), jnp.int32),
          pltpu.VMEM((CHUNK, width), jnp.float32),
          pltpu.VMEM_SHARED((NATOM, width), jnp.float32),
      ],
  )
  def k(pay_hbm, idx_hbm, z_hbm, out_hbm, idx_v, pbuf, acc, ):
    cid = lax.axis_index("c")
    sid = lax.axis_index("s")
    wid = sid * 2 + cid
    pltpu.sync_copy(z_hbm.at[pl.ds(sid * rows, rows)],
                    acc.at[pl.ds(sid * rows, rows)])
    pltpu.sync_copy(idx_hbm.at[wid], idx_v)
    plsc.subcore_barrier()

    def body(ci, carry):
      pltpu.sync_copy(pay_hbm.at[wid, ci], pbuf)
      pltpu.sync_copy(pbuf, acc.at[idx_v.at[ci]], add=True)
      return carry

    lax.fori_loop(0, SCH, body, 0)
    plsc.subcore_barrier()
    pltpu.sync_copy(acc.at[pl.ds(sid * rows, rows)],
                    out_hbm.at[cid, pl.ds(sid * rows, rows)])

  return k(pay4, idx3, zeros)


def _full(shape):
  return pl.BlockSpec(shape, lambda i: (0,) * len(shape))


def _init_body(zf_ref, r_ref, embed_ref, guess_ref, t_ref, h0p_ref, c0_ref):
  zf = zf_ref[...]
  ii = lax.broadcasted_iota(jnp.int32, (AT, MAX_Z + 1), 1).astype(jnp.float32)
  onehot = (zf == ii).astype(jnp.float32)
  h0 = jnp.dot(onehot, embed_ref[...], preferred_element_type=jnp.float32)
  c0 = jnp.dot(onehot, guess_ref[...], preferred_element_type=jnp.float32)
  z4 = jnp.zeros((AT, 4), jnp.float32)
  z8 = jnp.zeros((AT, 8), jnp.float32)
  t_ref[...] = jnp.concatenate([r_ref[...], z4[:, :1], h0, z4], axis=1)
  h0p_ref[...] = jnp.concatenate([h0, z8], axis=1)
  c0_ref[...] = c0


def _init_call(zf, r, embed, guess):
  grid = NATOM // AT
  return pl.pallas_call(
      _init_body,
      grid=(grid,),
      in_specs=[
          pl.BlockSpec((AT, 1), lambda i: (i, 0)),
          pl.BlockSpec((AT, 3), lambda i: (i, 0)),
          _full((MAX_Z + 1, N_EMBED)),
          _full((MAX_Z + 1, 1)),
      ],
      out_specs=[
          pl.BlockSpec((AT, 16), lambda i: (i, 0)),
          pl.BlockSpec((AT, 16), lambda i: (i, 0)),
          pl.BlockSpec((AT, 1), lambda i: (i, 0)),
      ],
      out_shape=[
          jax.ShapeDtypeStruct((NATOM, 16), jnp.float32),
          jax.ShapeDtypeStruct((NATOM, 16), jnp.float32),
          jax.ShapeDtypeStruct((NATOM, 1), jnp.float32),
      ],
  )(zf, r, embed, guess)


def _edge_inv_body(ts_ref, tt_ref, einv_ref, pinv_ref):
  i = pl.program_id(0)
  ts = ts_ref[...]
  tt = tt_ref[...]
  d = tt[:, 0:3] - ts[:, 0:3]
  dsq = jnp.sum(d * d, axis=1, keepdims=True)
  dR = jnp.sqrt(jnp.maximum(dsq, 0.0))
  dru = d / dR
  cutoff = 0.5 * (jnp.cos(jnp.pi * jnp.clip(dR, 0.0, R_CUT) / R_CUT) + 1.0)
  centers = lax.broadcasted_iota(jnp.int32, (1, N_RBF), 1).astype(
      jnp.float32) * (R_CUT / (N_RBF - 1))
  w = R_CUT / N_RBF
  rbf = jnp.exp(-((dR - centers) ** 2) / (2.0 * w * w)) * cutoff
  h0s = ts[:, 4:12]
  h0t = tt[:, 4:12]
  gid = i * ET + lax.broadcasted_iota(jnp.int32, (ET, 1), 0)
  valid = gid < NEDGE
  einv = jnp.concatenate(
      [h0s, h0t, rbf, dru, jnp.zeros((ET, 5), jnp.float32)], axis=1)
  einv_ref[...] = jnp.where(valid, einv, 0.0)
  outer = jnp.concatenate([rbf[:, r:r + 1] * h0t for r in range(8)], axis=1)
  pinv = jnp.concatenate(
      [jnp.ones((ET, 1), jnp.float32), rbf, h0t, outer,
       jnp.zeros((ET, 15), jnp.float32)], axis=1)
  pinv_ref[...] = jnp.where(valid, pinv, 0.0)


def _edge_inv_call(ts, tt):
  grid = EPAD // ET
  return pl.pallas_call(
      _edge_inv_body,
      grid=(grid,),
      in_specs=[
          pl.BlockSpec((ET, 16), lambda i: (i, 0)),
          pl.BlockSpec((ET, 16), lambda i: (i, 0)),
      ],
      out_specs=[
          pl.BlockSpec((ET, 32), lambda i: (i, 0)),
          pl.BlockSpec((ET, W_INV), lambda i: (i, 0)),
      ],
      out_shape=[
          jax.ShapeDtypeStruct((EPAD, 32), jnp.float32),
          jax.ShapeDtypeStruct((EPAD, W_INV), jnp.float32),
      ],
  )(ts, tt)


def _mlp_in_kernel(x, layers):
  n = len(layers)
  for i, (w_ref, b_ref) in enumerate(layers):
    x = jnp.dot(x, w_ref[...], preferred_element_type=jnp.float32) + b_ref[...]
    if i < n - 1:
      x = jnp.maximum(x, 0.0)
  return x


def _edge_iter_body(einv_ref, hs_ref, ht_ref, w1_ref, b1_ref, *rest):
  head_refs = rest[:18]
  pay_ref = rest[18]
  i = pl.program_id(0)
  einv = einv_ref[...]
  h0s = einv[:, 0:8]
  h0t = einv[:, 8:16]
  rbf = einv[:, 16:24]
  dru = einv[:, 24:27]
  hs = hs_ref[:, 0:8]
  ht = ht_ref[:, 0:8]
  h_all = jnp.concatenate([h0s, h0t, hs, ht], axis=1)
  x = jnp.concatenate(
      [h_all, rbf] + [rbf[:, r:r + 1] * h_all for r in range(8)], axis=1)
  y1 = jnp.dot(x, w1_ref[...], preferred_element_type=jnp.float32) + b1_ref[...]
  y1 = jnp.maximum(y1, 0.0)
  pieces = [ht] + [rbf[:, r:r + 1] * ht for r in range(8)]
  for hd in range(3):
    w2, b2, w3, b3, w4, b4 = head_refs[hd * 6:(hd + 1) * 6]
    y = y1[:, hd * 256:(hd + 1) * 256]
    y = _mlp_in_kernel(y, [(w2, b2), (w3, b3), (w4, b4)])
    pieces += [dru[:, xx:xx + 1] * y for xx in range(3)]
  pay = jnp.concatenate(pieces, axis=1)
  gid = i * ET + lax.broadcasted_iota(jnp.int32, (ET, 1), 0)
  pay_ref[...] = jnp.where(gid < NEDGE, pay, 0.0)


def _edge_iter_call(einv, hs, ht, w1, b1, head_ws):
  grid = EPAD // ET
  wspecs = [_full(w1.shape), _full(b1.shape)]
  for a in head_ws:
    wspecs.append(_full(a.shape))
  return pl.pallas_call(
      _edge_iter_body,
      grid=(grid,),
      in_specs=[
          pl.BlockSpec((ET, 32), lambda i: (i, 0)),
          pl.BlockSpec((ET, 16), lambda i: (i, 0)),
          pl.BlockSpec((ET, 16), lambda i: (i, 0)),
      ] + wspecs,
      out_specs=pl.BlockSpec((ET, W_ACC), lambda i: (i, 0)),
      out_shape=jax.ShapeDtypeStruct((EPAD, W_ACC), jnp.float32),
  )(einv, hs, ht, w1, b1, *head_ws)


def _node_body(acc_ref, inv_ref, hT_ref, h0p_ref, chg_ref, dip_ref, qp_ref,
               *rest):
  cu = [(rest[j * 2], rest[j * 2 + 1]) for j in range(4)]
  cr = [(rest[8 + j * 2], rest[8 + j * 2 + 1]) for j in range(4)]
  wd_ref, bd_ref, wq_ref, bq_ref = rest[16:20]
  hn_ref, chg_o, dip_o, qp_o = rest[20:24]
  acc = acc_ref[0] + acc_ref[1]
  inv = inv_ref[0] + inv_ref[1]
  h0 = h0p_ref[:, 0:8]
  h = hT_ref[:, 0:8]
  deg = inv[:, 0:1]
  s_rbf = inv[:, 1:9]
  s_h0t = inv[:, 9:17]
  s_ht = acc[:, 0:8]
  pieces = [h0 * deg, s_h0t, h * deg, s_ht, s_rbf]
  for r in range(8):
    pieces += [s_rbf[:, r:r + 1] * h0, inv[:, 17 + 8 * r:25 + 8 * r],
               s_rbf[:, r:r + 1] * h, acc[:, 8 + 8 * r:16 + 8 * r]]
  m_i = jnp.concatenate(pieces, axis=1)
  h_next = _mlp_in_kernel(m_i, cu)
  dc = _mlp_in_kernel(h_next, cr)
  chg_o[...] = chg_ref[...] + dc
  wd = wd_ref[...]
  bd = bd_ref[...]
  dipacc = acc[:, 72:96]
  dd = [jnp.dot(dipacc[:, xx * 8:(xx + 1) * 8], wd,
                preferred_element_type=jnp.float32) + bd for xx in range(3)]
  dip_o[...] = dip_ref[...] + jnp.concatenate(dd, axis=1)
  wq = wq_ref[...]
  bq = bq_ref[...]
  q1 = acc[:, 96:120]
  q2 = acc[:, 120:144]
  cols = []
  for xx in range(3):
    for yy in range(3):
      s = (q1[:, xx * 8:(xx + 1) * 8] * q2[:, yy * 8:(yy + 1) * 8]
           + q1[:, yy * 8:(yy + 1) * 8] * q2[:, xx * 8:(xx + 1) * 8])
      cols.append(jnp.dot(s, wq, preferred_element_type=jnp.float32) + bq)
  qp_o[...] = qp_ref[...] + jnp.concatenate(cols, axis=1)
  hn_ref[...] = jnp.concatenate(
      [h_next, jnp.zeros((AT, 8), jnp.float32)], axis=1)


def _node_call(acc, inv, hT, h0p, chg, dip, qp, weights):
  grid = NATOM // AT
  wspecs = [_full(a.shape) for a in weights]
  return pl.pallas_call(
      _node_body,
      grid=(grid,),
      in_specs=[
          pl.BlockSpec((2, AT, W_ACC), lambda i: (0, i, 0)),
          pl.BlockSpec((2, AT, W_INV), lambda i: (0, i, 0)),
          pl.BlockSpec((AT, 16), lambda i: (i, 0)),
          pl.BlockSpec((AT, 16), lambda i: (i, 0)),
          pl.BlockSpec((AT, 1), lambda i: (i, 0)),
          pl.BlockSpec((AT, 3), lambda i: (i, 0)),
          pl.BlockSpec((AT, 9), lambda i: (i, 0)),
      ] + wspecs,
      out_specs=[
          pl.BlockSpec((AT, 16), lambda i: (i, 0)),
          pl.BlockSpec((AT, 1), lambda i: (i, 0)),
          pl.BlockSpec((AT, 3), lambda i: (i, 0)),
          pl.BlockSpec((AT, 9), lambda i: (i, 0)),
      ],
      out_shape=[
          jax.ShapeDtypeStruct((NATOM, 16), jnp.float32),
          jax.ShapeDtypeStruct((NATOM, 1), jnp.float32),
          jax.ShapeDtypeStruct((NATOM, 3), jnp.float32),
          jax.ShapeDtypeStruct((NATOM, 9), jnp.float32),
      ],
  )(acc, inv, hT, h0p, chg, dip, qp, *weights)


def _mol_sum_body(chg_ref, mol_ref, tc_ref, err_ref, tq_ref, cnt_ref):
  i = pl.program_id(0)
  n = pl.num_programs(0)

  @pl.when(i == 0)
  def _():
    tq_ref[...] = jnp.zeros_like(tq_ref)
    cnt_ref[...] = jnp.zeros_like(cnt_ref)

  ii = lax.broadcasted_iota(jnp.int32, (AT, NMOL), 1).astype(jnp.float32)
  onehot = (mol_ref[...] == ii).astype(jnp.float32)
  tq_ref[...] += lax.dot_general(
      chg_ref[...], onehot, (((0,), (0,)), ((), ())),
      preferred_element_type=jnp.float32)
  cnt_ref[...] += lax.dot_general(
      jnp.ones((AT, 1), jnp.float32), onehot, (((0,), (0,)), ((), ())),
      preferred_element_type=jnp.float32)

  @pl.when(i == n - 1)
  def _():
    err_ref[...] = (tq_ref[...] - tc_ref[...]) / cnt_ref[...]


def _mol_sum_call(chg, molf, tcf):
  grid = NATOM // AT
  return pl.pallas_call(
      _mol_sum_body,
      grid=(grid,),
      in_specs=[
          pl.BlockSpec((AT, 1), lambda i: (i, 0)),
          pl.BlockSpec((AT, 1), lambda i: (i, 0)),
          _full((1, NMOL)),
      ],
      out_specs=_full((1, NMOL)),
      out_shape=jax.ShapeDtypeStruct((1, NMOL), jnp.float32),
      scratch_shapes=[
          pltpu.VMEM((1, NMOL), jnp.float32),
          pltpu.VMEM((1, NMOL), jnp.float32),
      ],
  )(chg, molf, tcf)


def _finalize_body(chg_ref, mol_ref, err_ref, qp_ref, chg_o, qp_o):
  ii = lax.broadcasted_iota(jnp.int32, (AT, NMOL), 1).astype(jnp.float32)
  onehot = (mol_ref[...] == ii).astype(jnp.float32)
  corr = lax.dot_general(onehot, err_ref[...], (((1,), (1,)), ((), ())),
                         preferred_element_type=jnp.float32)
  chg_o[...] = chg_ref[...] - corr
  q = qp_ref[...]
  tr = (q[:, 0:1] + q[:, 4:5] + q[:, 8:9]) / 3.0
  col = lax.broadcasted_iota(jnp.int32, (1, 9), 1)
  eye = (col % 4 == 0).astype(jnp.float32)
  qp_o[...] = q - tr * eye


def _finalize_call(chg, molf, err, qp):
  grid = NATOM // AT
  return pl.pallas_call(
      _finalize_body,
      grid=(grid,),
      in_specs=[
          pl.BlockSpec((AT, 1), lambda i: (i, 0)),
          pl.BlockSpec((AT, 1), lambda i: (i, 0)),
          _full((1, NMOL)),
          pl.BlockSpec((AT, 9), lambda i: (i, 0)),
      ],
      out_specs=[
          pl.BlockSpec((AT, 1), lambda i: (i, 0)),
          pl.BlockSpec((AT, 9), lambda i: (i, 0)),
      ],
      out_shape=[
          jax.ShapeDtypeStruct((NATOM, 1), jnp.float32),
          jax.ShapeDtypeStruct((NATOM, 9), jnp.float32),
      ],
  )(chg, molf, err, qp)


def _prep_update(layers):
  (w1, b1), rest = layers[0], layers[1:]
  w1p = w1[jnp.asarray(_PERM), :]
  out = [w1p, b1.reshape(1, -1)]
  for w, b in rest:
    out += [w, b.reshape(1, -1)]
  return out


def kernel(Z, R, e_source, e_target, molecule_ind, total_charge, params):
  es = e_source.astype(jnp.int32)
  et = e_target.astype(jnp.int32)
  pad = jnp.zeros((EPAD - NEDGE,), jnp.int32)
  esp = jnp.concatenate([es, pad])
  etp = jnp.concatenate([et, pad])
  both = jnp.concatenate([esp, etp])

  zf = Z.astype(jnp.float32).reshape(NATOM, 1)
  molf = molecule_ind.astype(jnp.float32).reshape(NATOM, 1)
  tcf = total_charge.astype(jnp.float32).reshape(1, NMOL)

  T, h0p, charge = _init_call(zf, R, params['embed'], params['guess'])

  g = _sc_gather(T, both)
  einv, pinv = _edge_inv_call(g[:EPAD], g[EPAD:])
  inv = _sc_scatter(pinv, esp, W_INV)

  dipole = jnp.zeros((NATOM, 3), jnp.float32)
  qpole = jnp.zeros((NATOM, 9), jnp.float32)

  cu_w = [_prep_update(params['charge_update_%d' % i]) for i in range(N_MESSAGE)]
  node_w = []
  edge_w = []
  for i in range(N_MESSAGE):
    dW = _prep_update(params['dipole_update_%d' % i])
    q1W = _prep_update(params['qpole1_update_%d' % i])
    q2W = _prep_update(params['qpole2_update_%d' % i])
    w1c = jnp.concatenate([dW[0], q1W[0], q2W[0]], axis=1)
    b1c = jnp.concatenate([dW[1], q1W[1], q2W[1]], axis=1)
    heads = dW[2:] + q1W[2:] + q2W[2:]
    edge_w.append((w1c, b1c, heads))
    cr = params['charge_readout_%d' % i]
    nw = list(cu_w[i])
    for w, b in cr:
      nw += [w, b.reshape(1, -1)]
    nw += [params['dipole_readout_%d' % i][0][0],
           params['dipole_readout_%d' % i][0][1].reshape(1, 1),
           params['qpole_readout_%d' % i][0][0],
           params['qpole_readout_%d' % i][0][1].reshape(1, 1)]
    node_w.append(nw)

  hT = h0p
  h_list = [h0p[:, :N_EMBED]]
  for i in range(N_MESSAGE):
    gh = _sc_gather(hT, both)
    w1c, b1c, heads = edge_w[i]
    pay = _edge_iter_call(einv, gh[:EPAD], gh[EPAD:], w1c, b1c, heads)
    acc = _sc_scatter(pay, esp, W_ACC)
    hT, charge, dipole, qpole = _node_call(
        acc, inv, hT, h0p, charge, dipole, qpole, node_w[i])
    h_list.append(hT[:, :N_EMBED])

  err = _mol_sum_call(charge, molf, tcf)
  charge, qpole = _finalize_call(charge, molf, err, qpole)

  return charge, dipole, qpole.reshape(NATOM, 3, 3), h_list
